# baseline (speedup 1.0000x reference)
import functools

import jax
import jax.numpy as jnp
from jax import lax
from jax.experimental import pallas as pl
from jax.experimental.pallas import tpu as pltpu
from jax.experimental.pallas import tpu_sc as plsc

ICO_N_IN = 10242
N_OUT = 40962
C = 256
LANES = 16
CH = 64
T_FULL = N_OUT // CH
TAIL = N_OUT - T_FULL * CH
NW = 32
KPW = T_FULL // NW
MAXK = KPW + 1
PAD_CHUNKS = MAXK * NW


def _build(mesh, scale):
    @functools.partial(
        pl.kernel,
        out_type=jax.ShapeDtypeStruct((N_OUT, C), jnp.float32),
        mesh=mesh,
        scratch_types=[
            pltpu.VMEM((MAXK * CH,), jnp.int32),
            pltpu.VMEM((MAXK * CH,), jnp.int32),
            pltpu.VMEM((CH, C), jnp.float32),
            pltpu.VMEM((CH, C), jnp.float32),
            pltpu.VMEM((CH, C), jnp.float32),
            pltpu.VMEM((CH, C), jnp.float32),
            pltpu.VMEM((CH, C), jnp.float32),
            pltpu.VMEM((CH, C), jnp.float32),
            pltpu.VMEM((LANES, C), jnp.float32),
            pltpu.VMEM((LANES,), jnp.int32),
            pltpu.SemaphoreType.DMA,
            pltpu.SemaphoreType.DMA,
            pltpu.SemaphoreType.DMA,
            pltpu.SemaphoreType.DMA,
            pltpu.SemaphoreType.DMA,
        ],
    )
    def k(x_hbm, idx0_hbm, idx1_hbm, out_hbm,
          i0, i1, b0a, b0b, b1a, b1b, ova, ovb, tl, tidx,
          g0, g1, st0, st1, gi):
        w = lax.axis_index("s") * 2 + lax.axis_index("c")
        start = KPW * w
        last = w == NW - 1

        ci0 = pltpu.async_copy(idx0_hbm.at[pl.ds(start * CH, MAXK * CH)], i0, gi)
        ci1 = pltpu.async_copy(idx1_hbm.at[pl.ds(start * CH, MAXK * CH)], i1, gi)
        ci0.wait()
        ci1.wait()

        b0 = (b0a, b0b)
        b1 = (b1a, b1b)
        ov = (ova, ovb)
        g = (g0, g1)
        st = (st0, st1)

        def gather_pair(kk):
            s = kk & 1
            isl = pl.ds(kk * CH, CH)
            return (pltpu.make_async_copy(x_hbm.at[i0.at[isl]], b0[s], g[s]),
                    pltpu.make_async_copy(x_hbm.at[i1.at[isl]], b1[s], g[s]))

        def fire_gathers(kk):
            c0, c1 = gather_pair(kk)
            c0.start()
            c1.start()

        def store_copy(kk):
            s = kk & 1
            return pltpu.make_async_copy(
                ov[s], out_hbm.at[pl.ds((start + kk) * CH, CH)], st[s])

        fire_gathers(0)
        for kk in range(KPW):
            s = kk & 1
            if kk + 1 < KPW:
                fire_gathers(kk + 1)
            else:
                @pl.when(last)
                def _():
                    fire_gathers(KPW)

            c0, c1 = gather_pair(kk)
            c0.wait()
            c1.wait()
            if kk >= 2:
                store_copy(kk - 2).wait()

            @plsc.parallel_loop(0, CH * (C // LANES), step=1, unroll=16)
            def _(q, s=s):
                i = q >> 4
                sl = pl.ds((q & 15) * LANES, LANES)
                ov[s][i, sl] = (b0[s][i, sl] + b1[s][i, sl]) * scale

            store_copy(kk).start()

        @pl.when(last)
        def _():
            s = KPW & 1
            c0, c1 = gather_pair(KPW)
            c0.wait()
            c1.wait()
            for r in range(TAIL):
                for j in range(C // LANES):
                    sl = pl.ds(j * LANES, LANES)
                    tl[r, sl] = (b0[s][r, sl] + b1[s][r, sl]) * scale

            def rep_body(r, _):
                for j in range(C // LANES):
                    sl = pl.ds(j * LANES, LANES)
                    tl[r, sl] = tl[r & 1, sl]
                return 0

            lax.fori_loop(TAIL, LANES, rep_body, 0)
            rows = T_FULL * CH + (lax.iota(jnp.int32, LANES) & (TAIL - 1))
            tidx[...] = rows
            pltpu.async_copy(tl, out_hbm.at[tidx], gi).wait()

        store_copy(KPW - 2).wait()
        store_copy(KPW - 1).wait()

    return k


def kernel(x, batch_size, reverse_hex):
    del batch_size
    rh = reverse_hex.astype(jnp.int32)
    pad = PAD_CHUNKS * CH - N_OUT
    idx0 = jnp.pad(rh[:, 0], (0, pad))
    idx1 = jnp.pad(rh[:, 1], (0, pad))
    scale = 1.0 / (x.shape[0] // ICO_N_IN)
    mesh = plsc.VectorSubcoreMesh(core_axis_name="c", subcore_axis_name="s")
    return _build(mesh, scale)(x, idx0, idx1)


# speedup vs baseline: 1.0215x; 1.0215x over previous
import functools

import jax
import jax.numpy as jnp
from jax import lax
from jax.experimental import pallas as pl
from jax.experimental.pallas import tpu as pltpu
from jax.experimental.pallas import tpu_sc as plsc

ICO_N_IN = 10242
N_OUT = 40962
C = 256
LANES = 16
CH = 64
T_FULL = N_OUT // CH
TAIL = N_OUT - T_FULL * CH
NW = 32
KPW = T_FULL // NW
MAXK = KPW + 1
PAD_CHUNKS = MAXK * NW


def _build(mesh, scale):
    @functools.partial(
        pl.kernel,
        out_type=jax.ShapeDtypeStruct((N_OUT, C), jnp.float32),
        mesh=mesh,
        scratch_types=[
            pltpu.VMEM((MAXK * CH,), jnp.int32),
            pltpu.VMEM((MAXK * CH,), jnp.int32),
            pltpu.VMEM((CH, C), jnp.float32),
            pltpu.VMEM((CH, C), jnp.float32),
            pltpu.VMEM((CH, C), jnp.float32),
            pltpu.VMEM((CH, C), jnp.float32),
            pltpu.VMEM((CH, C), jnp.float32),
            pltpu.VMEM((CH, C), jnp.float32),
            pltpu.VMEM((LANES, C), jnp.float32),
            pltpu.VMEM((LANES,), jnp.int32),
            pltpu.SemaphoreType.DMA,
            pltpu.SemaphoreType.DMA,
            pltpu.SemaphoreType.DMA,
            pltpu.SemaphoreType.DMA,
            pltpu.SemaphoreType.DMA,
        ],
    )
    def k(x_hbm, idx0_hbm, idx1_hbm, out_hbm,
          i0, i1, b0a, b0b, b1a, b1b, ova, ovb, tl, tidx,
          g0, g1, st0, st1, gi):
        w = lax.axis_index("s") * 2 + lax.axis_index("c")
        start = KPW * w
        last = w == NW - 1

        ci0 = pltpu.async_copy(idx0_hbm.at[pl.ds(start * CH, MAXK * CH)], i0, gi)
        ci1 = pltpu.async_copy(idx1_hbm.at[pl.ds(start * CH, MAXK * CH)], i1, gi)
        ci0.wait()
        ci1.wait()

        b0 = (b0a, b0b)
        b1 = (b1a, b1b)
        ov = (ova, ovb)
        g = (g0, g1)
        st = (st0, st1)

        def gather_pair(kk):
            s = kk & 1
            isl = pl.ds(kk * CH, CH)
            return (pltpu.make_async_copy(x_hbm.at[i0.at[isl]], b0[s], g[s]),
                    pltpu.make_async_copy(x_hbm.at[i1.at[isl]], b1[s], g[s]))

        def fire_gathers(kk):
            c0, c1 = gather_pair(kk)
            c0.start()
            c1.start()

        def store_copy(kk):
            s = kk & 1
            return pltpu.make_async_copy(
                ov[s], out_hbm.at[pl.ds((start + kk) * CH, CH)], st[s])

        fire_gathers(0)
        for kk in range(KPW):
            s = kk & 1
            if kk + 1 < KPW:
                fire_gathers(kk + 1)
            else:
                @pl.when(last)
                def _():
                    fire_gathers(KPW)

            c0, c1 = gather_pair(kk)
            c0.wait()
            c1.wait()
            if kk >= 2:
                store_copy(kk - 2).wait()

            @plsc.parallel_loop(0, CH * (C // LANES), step=1, unroll=8)
            def _(q, s=s):
                i = q >> 4
                sl = pl.ds((q & 15) * LANES, LANES)
                ov[s][i, sl] = (b0[s][i, sl] + b1[s][i, sl]) * scale

            store_copy(kk).start()

        @pl.when(last)
        def _():
            s = KPW & 1
            c0, c1 = gather_pair(KPW)
            c0.wait()
            c1.wait()
            for r in range(TAIL):
                for j in range(C // LANES):
                    sl = pl.ds(j * LANES, LANES)
                    tl[r, sl] = (b0[s][r, sl] + b1[s][r, sl]) * scale

            def rep_body(r, _):
                for j in range(C // LANES):
                    sl = pl.ds(j * LANES, LANES)
                    tl[r, sl] = tl[r & 1, sl]
                return 0

            lax.fori_loop(TAIL, LANES, rep_body, 0)
            rows = T_FULL * CH + (lax.iota(jnp.int32, LANES) & (TAIL - 1))
            tidx[...] = rows
            pltpu.async_copy(tl, out_hbm.at[tidx], gi).wait()

        store_copy(KPW - 2).wait()
        store_copy(KPW - 1).wait()

    return k


def kernel(x, batch_size, reverse_hex):
    del batch_size
    rh = reverse_hex.astype(jnp.int32)
    pad = PAD_CHUNKS * CH - N_OUT
    idx0 = jnp.pad(rh[:, 0], (0, pad))
    idx1 = jnp.pad(rh[:, 1], (0, pad))
    scale = 1.0 / (x.shape[0] // ICO_N_IN)
    mesh = plsc.VectorSubcoreMesh(core_axis_name="c", subcore_axis_name="s")
    return _build(mesh, scale)(x, idx0, idx1)
